# Initial kernel scaffold; baseline (speedup 1.0000x reference)
import jax
import jax.numpy as jnp
from jax.experimental import pallas as pl


def kernel(features, edge_index, W, b, w_a1, b_a1, w_a2, b_a2):
    raise NotImplementedError("write your pallas kernel here")


# speedup vs baseline: 15.4494x; 15.4494x over previous
import functools

import jax
import jax.numpy as jnp
from jax import lax
from jax.experimental import pallas as pl
from jax.experimental.pallas import tpu as pltpu
from jax.experimental.pallas import tpu_sc as plsc

N = 10000
D = 128
E = 320000
NTILES = 32
CHUNK = 128
GRP = 8
NGRP = 10
NCHUNKS = GRP * NGRP
EPT = NCHUNKS * CHUNK
EPAD = NTILES * EPT
NPAD = 10016
NROWS = 80
NSP = 10112
BLK = 400


def _prep_body(x_ref, w_ref, b_ref, wa_ref, ba_ref, feat_ref, a_ref):
    feat = jnp.dot(x_ref[...], w_ref[...],
                   preferred_element_type=jnp.float32) + b_ref[...]
    feat_ref[...] = feat
    a_ref[...] = jnp.dot(feat, wa_ref[...],
                         preferred_element_type=jnp.float32) + ba_ref[...]


def _prep(features, W, b2d, wa, ba):
    return pl.pallas_call(
        _prep_body,
        grid=(N // BLK,),
        in_specs=[
            pl.BlockSpec((BLK, D), lambda i: (i, 0)),
            pl.BlockSpec((D, D), lambda i: (0, 0)),
            pl.BlockSpec((1, D), lambda i: (0, 0)),
            pl.BlockSpec((D, 2), lambda i: (0, 0)),
            pl.BlockSpec((1, 2), lambda i: (0, 0)),
        ],
        out_specs=[
            pl.BlockSpec((BLK, D), lambda i: (i, 0)),
            pl.BlockSpec((BLK, 2), lambda i: (i, 0)),
        ],
        out_shape=[
            jax.ShapeDtypeStruct((N, D), jnp.float32),
            jax.ShapeDtypeStruct((N, 2), jnp.float32),
        ],
    )(features, W, b2d, wa, ba)


def _edge_body(src_hbm, dst_hbm, a1_hbm, a2_hbm, feat_hbm, zfeat_hbm,
               numer_out, denom_out,
               a1_v, a2_v, src_v, dst_v, ex_v, den_v, row_v, numer_sh, sem):
    cid = lax.axis_index("c")
    sid = lax.axis_index("s")
    tid = cid * 16 + sid

    pltpu.sync_copy(a1_hbm, a1_v)
    pltpu.sync_copy(a2_hbm, a2_v)

    def zden(i, carry):
        for c in range(8):
            den_v[i, pl.ds(c * 16, 16)] = jnp.zeros((16,), jnp.float32)
        return carry
    lax.fori_loop(0, NROWS, zden, 0)

    rows_per_tile = NSP // 16
    pltpu.sync_copy(zfeat_hbm.at[pl.ds(sid * rows_per_tile, rows_per_tile)],
                    numer_sh.at[pl.ds(sid * rows_per_tile, rows_per_tile)])
    plsc.subcore_barrier()

    def group_body(gi, carry):
        pltpu.sync_copy(src_hbm.at[tid, pl.ds(gi * GRP, GRP)], src_v)
        pltpu.sync_copy(dst_hbm.at[tid, pl.ds(gi * GRP, GRP)], dst_v)

        def chunk_body(g, c1):
            def vec_body(j, c2):
                s = src_v[g, pl.ds(j * 16, 16)]
                dn = dst_v[g, pl.ds(j * 16, 16)]
                a1g = plsc.load_gather(a1_v, [s])
                a2g = plsc.load_gather(a2_v, [dn])
                v = a1g + a2g
                v = jnp.where(v >= 0.0, v, 0.01 * v)
                e = jnp.exp(v)
                ex_v[pl.ds(j * 16, 16)] = e
                plsc.addupdate_scatter(
                    den_v,
                    [lax.shift_right_logical(s, 7), lax.bitwise_and(s, 127)],
                    e)
                return c2
            lax.fori_loop(0, CHUNK // 16, vec_body, 0)

            pltpu.async_copy(feat_hbm.at[dst_v.at[g]], row_v, sem).wait()

            def row_body(j, c2):
                ev = plsc.load_gather(
                    ex_v, [jnp.broadcast_to(j, (16,)).astype(jnp.int32)])
                for c in range(D // 16):
                    row_v[j, pl.ds(c * 16, 16)] = (
                        row_v[j, pl.ds(c * 16, 16)] * ev)
                return c2
            lax.fori_loop(0, CHUNK, row_body, 0)

            pltpu.sync_copy(row_v, numer_sh.at[src_v.at[g]], add=True)
            return c1
        lax.fori_loop(0, GRP, chunk_body, 0)
        return carry
    lax.fori_loop(0, NGRP, group_body, 0)

    plsc.subcore_barrier()
    rpt = NSP // 16
    pltpu.sync_copy(numer_sh.at[pl.ds(sid * rpt, rpt)],
                    numer_out.at[cid, pl.ds(sid * rpt, rpt)])
    pltpu.sync_copy(den_v, denom_out.at[tid])


_edge_call = functools.partial(
    pl.kernel,
    mesh=plsc.VectorSubcoreMesh(core_axis_name="c", subcore_axis_name="s"),
    compiler_params=pltpu.CompilerParams(needs_layout_passes=False),
    out_type=[
        jax.ShapeDtypeStruct((2, NSP, D), jnp.float32),
        jax.ShapeDtypeStruct((NTILES, NROWS, 128), jnp.float32),
    ],
    scratch_types=[
        pltpu.VMEM((NPAD,), jnp.float32),
        pltpu.VMEM((NPAD,), jnp.float32),
        pltpu.VMEM((GRP, CHUNK), jnp.int32),
        pltpu.VMEM((GRP, CHUNK), jnp.int32),
        pltpu.VMEM((CHUNK,), jnp.float32),
        pltpu.VMEM((NROWS, 128), jnp.float32),
        pltpu.VMEM((CHUNK, D), jnp.float32),
        pltpu.VMEM_SHARED((NSP, D), jnp.float32),
        pltpu.SemaphoreType.DMA,
    ],
)(_edge_body)


def _finish_body(n_ref, d_ref, o_ref):
    nsum = n_ref[0] + n_ref[1]
    d = jnp.sum(d_ref[...], axis=1)
    d = jnp.where(d == 0.0, 1.0, d)
    o_ref[...] = nsum / d[:, None]


def _finish(numer, dsum):
    return pl.pallas_call(
        _finish_body,
        grid=(N // BLK,),
        in_specs=[
            pl.BlockSpec((2, BLK, D), lambda i: (0, i, 0)),
            pl.BlockSpec((BLK, NTILES), lambda i: (i, 0)),
        ],
        out_specs=pl.BlockSpec((BLK, D), lambda i: (i, 0)),
        out_shape=jax.ShapeDtypeStruct((N, D), jnp.float32),
    )(numer, dsum)


def kernel(features, edge_index, W, b, w_a1, b_a1, w_a2, b_a2):
    src = edge_index[0].astype(jnp.int32)
    dst = edge_index[1].astype(jnp.int32)
    npe = EPAD - E
    src_p = jnp.concatenate(
        [src, jnp.full((npe,), N, jnp.int32)]).reshape(NTILES, NCHUNKS, CHUNK)
    dst_p = jnp.concatenate(
        [dst, jnp.zeros((npe,), jnp.int32)]).reshape(NTILES, NCHUNKS, CHUNK)
    wa = jnp.stack([w_a1, w_a2], axis=1)
    ba = jnp.stack([b_a1, b_a2]).reshape(1, 2)
    feat, a = _prep(features, W, b.reshape(1, D), wa, ba)
    a_pad = jnp.pad(a, ((0, NPAD - N), (0, 0)), constant_values=-1e30)
    a1 = a_pad[:, 0]
    a2 = a_pad[:, 1]
    zfeat = jnp.zeros((NSP, D), jnp.float32)
    numer, denom = _edge_call(src_p, dst_p, a1, a2, feat, zfeat)
    dsum = denom.reshape(NTILES, NROWS * 128)[:, :N].T
    return _finish(numer, dsum)


# speedup vs baseline: 18.9317x; 1.2254x over previous
import functools

import jax
import jax.numpy as jnp
from jax import lax
from jax.experimental import pallas as pl
from jax.experimental.pallas import tpu as pltpu
from jax.experimental.pallas import tpu_sc as plsc

N = 10000
D = 128
E = 320000
NTILES = 32
CHUNK = 128
CPG = 8
NGRP = 10
NCHUNKS = CPG * NGRP
EPG = CPG * CHUNK
EPT = NCHUNKS * CHUNK
EPAD = NTILES * EPT
NPAD = 10016
NROWS = 79
NSP = 10112
BLK = 400


def _prep_body(x_ref, w_ref, b_ref, wa_ref, ba_ref, feat_ref, a_ref):
    feat = jnp.dot(x_ref[...], w_ref[...],
                   preferred_element_type=jnp.float32) + b_ref[...]
    feat_ref[...] = feat
    a_ref[...] = jnp.dot(feat, wa_ref[...],
                         preferred_element_type=jnp.float32) + ba_ref[...]


def _prep(features, W, b2d, wa, ba):
    return pl.pallas_call(
        _prep_body,
        grid=(N // BLK,),
        in_specs=[
            pl.BlockSpec((BLK, D), lambda i: (i, 0)),
            pl.BlockSpec((D, D), lambda i: (0, 0)),
            pl.BlockSpec((1, D), lambda i: (0, 0)),
            pl.BlockSpec((D, 2), lambda i: (0, 0)),
            pl.BlockSpec((1, 2), lambda i: (0, 0)),
        ],
        out_specs=[
            pl.BlockSpec((BLK, D), lambda i: (i, 0)),
            pl.BlockSpec((BLK, 2), lambda i: (i, 0)),
        ],
        out_shape=[
            jax.ShapeDtypeStruct((N, D), jnp.float32),
            jax.ShapeDtypeStruct((N, 2), jnp.float32),
        ],
    )(features, W, b2d, wa, ba)


def _ex_body(src_hbm, dst_hbm, a1_hbm, a2_hbm, ex_out, denom_out,
             a1_v, a2_v, src_v, dst_v, ex_v, den_v):
    cid = lax.axis_index("c")
    sid = lax.axis_index("s")
    tid = cid * 16 + sid

    pltpu.sync_copy(a1_hbm, a1_v)
    pltpu.sync_copy(a2_hbm, a2_v)

    def zden(i, carry):
        for c in range(8):
            den_v[i, pl.ds(c * 16, 16)] = jnp.zeros((16,), jnp.float32)
        return carry
    lax.fori_loop(0, NROWS, zden, 0)

    def group_body(gi, carry):
        pltpu.sync_copy(src_hbm.at[tid, pl.ds(gi * CPG, CPG)], src_v)
        pltpu.sync_copy(dst_hbm.at[tid, pl.ds(gi * CPG, CPG)], dst_v)

        def vec_body(j, c2):
            r = lax.div(j, CHUNK // 16)
            o = lax.rem(j, CHUNK // 16) * 16
            s = src_v[r, pl.ds(o, 16)]
            dn = dst_v[r, pl.ds(o, 16)]
            v = plsc.load_gather(a1_v, [s]) + plsc.load_gather(a2_v, [dn])
            v = jnp.where(v >= 0.0, v, 0.01 * v)
            e = jnp.exp(v)
            ex_v[pl.ds(j * 16, 16)] = e
            plsc.addupdate_scatter(
                den_v,
                [lax.shift_right_logical(s, 7), lax.bitwise_and(s, 127)],
                e)
            return c2
        lax.fori_loop(0, EPG // 16, vec_body, 0)

        pltpu.sync_copy(ex_v, ex_out.at[tid, gi])
        return carry
    lax.fori_loop(0, NGRP, group_body, 0)

    pltpu.sync_copy(den_v, denom_out.at[tid])


_ex_call = functools.partial(
    pl.kernel,
    mesh=plsc.VectorSubcoreMesh(core_axis_name="c", subcore_axis_name="s"),
    compiler_params=pltpu.CompilerParams(needs_layout_passes=False),
    out_type=[
        jax.ShapeDtypeStruct((NTILES, NGRP, EPG), jnp.float32),
        jax.ShapeDtypeStruct((NTILES, NROWS, 128), jnp.float32),
    ],
    scratch_types=[
        pltpu.VMEM((NPAD,), jnp.float32),
        pltpu.VMEM((NPAD,), jnp.float32),
        pltpu.VMEM((CPG, CHUNK), jnp.int32),
        pltpu.VMEM((CPG, CHUNK), jnp.int32),
        pltpu.VMEM((EPG,), jnp.float32),
        pltpu.VMEM((NROWS, 128), jnp.float32),
    ],
)(_ex_body)


def _msg_body(src_hbm, dst_hbm, ex_hbm, feat_hbm, zfeat_hbm, numer_out,
              src_v, dst_v, ex_v, row_a, row_b,
              numer_sh, gsem_a, gsem_b, ssem_a, ssem_b):
    cid = lax.axis_index("c")
    sid = lax.axis_index("s")
    tid = cid * 16 + sid
    rows = (row_a, row_b)
    gsems = (gsem_a, gsem_b)
    ssems = (ssem_a, ssem_b)

    rpt = NSP // 16
    pltpu.sync_copy(zfeat_hbm.at[pl.ds(sid * rpt, rpt)],
                    numer_sh.at[pl.ds(sid * rpt, rpt)])
    plsc.subcore_barrier()

    def _scatter(b, g):
        return pltpu.make_async_copy(
            rows[b], numer_sh.at[src_v.at[g]], ssems[b])

    def _gather(b, g):
        return pltpu.make_async_copy(
            feat_hbm.at[dst_v.at[g]], rows[b], gsems[b])

    def group_body(gi, carry):
        @pl.when(gi > 0)
        def _():
            _scatter(0, CPG - 2).wait()
            _scatter(1, CPG - 1).wait()
        pltpu.sync_copy(src_hbm.at[tid, pl.ds(gi * CPG, CPG)], src_v)
        pltpu.sync_copy(dst_hbm.at[tid, pl.ds(gi * CPG, CPG)], dst_v)
        pltpu.sync_copy(ex_hbm.at[tid, gi], ex_v)
        _gather(0, 0).start()

        def chunk_body(p, c1):
            for b in range(2):
                g = 2 * p + b
                _gather(b, g).wait()
                if b == 0:
                    @pl.when(p > 0)
                    def _():
                        _scatter(1, g).wait()
                    _gather(1, g + 1).start()
                else:
                    @pl.when(p < CPG // 2 - 1)
                    def _():
                        _scatter(0, g).wait()
                        _gather(0, g + 1).start()

                rv = rows[b]

                @plsc.parallel_loop(0, CHUNK, 1, unroll=4)
                def _(j):
                    ev = plsc.load_gather(
                        ex_v,
                        [jnp.broadcast_to(g * CHUNK + j, (16,)
                                          ).astype(jnp.int32)])
                    for c in range(D // 16):
                        rv[j, pl.ds(c * 16, 16)] = (
                            rv[j, pl.ds(c * 16, 16)] * ev)

                pltpu.async_copy(rv, numer_sh.at[src_v.at[g]],
                                 ssems[b], add=True)
            return c1
        lax.fori_loop(0, CPG // 2, chunk_body, 0)
        return carry
    lax.fori_loop(0, NGRP, group_body, 0)

    _scatter(0, CPG - 2).wait()
    _scatter(1, CPG - 1).wait()

    plsc.subcore_barrier()
    pltpu.sync_copy(numer_sh.at[pl.ds(sid * rpt, rpt)],
                    numer_out.at[cid, pl.ds(sid * rpt, rpt)])


_msg_call = functools.partial(
    pl.kernel,
    mesh=plsc.VectorSubcoreMesh(core_axis_name="c", subcore_axis_name="s"),
    compiler_params=pltpu.CompilerParams(needs_layout_passes=False),
    out_type=jax.ShapeDtypeStruct((2, NSP, D), jnp.float32),
    scratch_types=[
        pltpu.VMEM((CPG, CHUNK), jnp.int32),
        pltpu.VMEM((CPG, CHUNK), jnp.int32),
        pltpu.VMEM((EPG,), jnp.float32),
        pltpu.VMEM((CHUNK, D), jnp.float32),
        pltpu.VMEM((CHUNK, D), jnp.float32),
        pltpu.VMEM_SHARED((NSP, D), jnp.float32),
        pltpu.SemaphoreType.DMA,
        pltpu.SemaphoreType.DMA,
        pltpu.SemaphoreType.DMA,
        pltpu.SemaphoreType.DMA,
    ],
)(_msg_body)


def _finish_body(n_ref, d_ref, o_ref):
    nsum = n_ref[0] + n_ref[1]
    d = jnp.sum(d_ref[...], axis=1)
    d = jnp.where(d == 0.0, 1.0, d)
    o_ref[...] = nsum / d[:, None]


def _finish(numer, dsum):
    return pl.pallas_call(
        _finish_body,
        grid=(N // BLK,),
        in_specs=[
            pl.BlockSpec((2, BLK, D), lambda i: (0, i, 0)),
            pl.BlockSpec((BLK, NTILES), lambda i: (i, 0)),
        ],
        out_specs=pl.BlockSpec((BLK, D), lambda i: (i, 0)),
        out_shape=jax.ShapeDtypeStruct((N, D), jnp.float32),
    )(numer, dsum)


def kernel(features, edge_index, W, b, w_a1, b_a1, w_a2, b_a2):
    src = edge_index[0].astype(jnp.int32)
    dst = edge_index[1].astype(jnp.int32)
    npe = EPAD - E
    src_p = jnp.concatenate(
        [src, jnp.full((npe,), N, jnp.int32)]).reshape(NTILES, NCHUNKS, CHUNK)
    dst_p = jnp.concatenate(
        [dst, jnp.zeros((npe,), jnp.int32)]).reshape(NTILES, NCHUNKS, CHUNK)
    wa = jnp.stack([w_a1, w_a2], axis=1)
    ba = jnp.stack([b_a1, b_a2]).reshape(1, 2)
    feat, a = _prep(features, W, b.reshape(1, D), wa, ba)
    a_pad = jnp.pad(a, ((0, NPAD - N), (0, 0)), constant_values=-1e30)
    a1 = a_pad[:, 0]
    a2 = a_pad[:, 1]
    ex, denom = _ex_call(src_p, dst_p, a1, a2)
    zfeat = jnp.zeros((NSP, D), jnp.float32)
    numer = _msg_call(src_p, dst_p, ex, feat, zfeat)
    dsum = denom.reshape(NTILES, NROWS * 128)[:, :N].T
    return _finish(numer, dsum)
